# baseline (device time: 51986 ns/iter reference)
import jax
import jax.numpy as jnp
from jax import lax
from jax.experimental import pallas as pl
from jax.experimental.pallas import tpu as pltpu


def _body(q_ref, k_ref, v_ref, out_ref,
          o_send, o_recv, st_send, st_recv,
          so_sems, ss_sems, ro_sems, rs_sems):
    nb = o_send.shape[0]
    d = q_ref.shape[-1]
    scale = d ** -0.5
    bi = pl.program_id(0)

    my_x = lax.axis_index("x")
    my_y = lax.axis_index("y")
    nbr = (my_x, 1 - my_y)

    @pl.when(bi == 0)
    def _barrier():
        barrier = pltpu.get_barrier_semaphore()
        pl.semaphore_signal(
            barrier, inc=1, device_id=nbr,
            device_id_type=pl.DeviceIdType.MESH,
        )
        pl.semaphore_wait(barrier, 1)

    q = jnp.swapaxes(q_ref[0].astype(jnp.bfloat16), 0, 1)
    k = jnp.swapaxes(k_ref[0].astype(jnp.bfloat16), 0, 1)
    s = lax.dot_general(
        q, k, (((2,), (2,)), ((0,), (0,))),
        preferred_element_type=jnp.float32,
    ) * scale
    m = jnp.max(s, axis=-1, keepdims=True)
    p = jnp.exp(s - m)
    lsum = jnp.sum(p, axis=-1, keepdims=True)
    v = jnp.swapaxes(v_ref[0].astype(jnp.bfloat16), 0, 1)
    o = lax.dot_general(
        p.astype(jnp.bfloat16), v, (((2,), (1,)), ((0,), (0,))),
        preferred_element_type=jnp.float32,
    )
    o_send[bi] = o.astype(jnp.bfloat16)
    st_send[bi, 0] = m[..., 0]
    st_send[bi, 1] = lsum[..., 0]

    def _rdmas(idx):
        rdma_o = pltpu.make_async_remote_copy(
            src_ref=o_send.at[idx], dst_ref=o_recv.at[idx],
            send_sem=so_sems.at[idx], recv_sem=ro_sems.at[idx],
            device_id=nbr, device_id_type=pl.DeviceIdType.MESH,
        )
        rdma_st = pltpu.make_async_remote_copy(
            src_ref=st_send.at[idx], dst_ref=st_recv.at[idx],
            send_sem=ss_sems.at[idx], recv_sem=rs_sems.at[idx],
            device_id=nbr, device_id_type=pl.DeviceIdType.MESH,
        )
        return rdma_o, rdma_st

    rdma_o, rdma_st = _rdmas(bi)
    rdma_o.start()
    rdma_st.start()

    @pl.when(bi == nb - 1)
    def _drain_and_combine():
        for j in range(nb):
            ro, rs = _rdmas(j)
            ro.wait()
            rs.wait()

        m_loc = st_send[:, 0]
        l_loc = st_send[:, 1]
        m_nbr = st_recv[:, 0]
        l_nbr = st_recv[:, 1]
        m_new = jnp.maximum(m_loc, m_nbr)
        a_loc = jnp.exp(m_loc - m_new)
        a_nbr = jnp.exp(m_nbr - m_new)
        l_new = a_loc * l_loc + a_nbr * l_nbr
        o_loc = o_send[...].astype(jnp.float32)
        o_nbr = o_recv[...].astype(jnp.float32)
        num = a_loc[..., None] * o_loc + a_nbr[..., None] * o_nbr
        res = num / l_new[..., None]
        out_ref[...] = jnp.swapaxes(res, 1, 2)


def kernel(Q, K, V):
    b, sq, h, d = Q.shape
    skv = K.shape[1]

    return pl.pallas_call(
        _body,
        grid=(b,),
        in_specs=[
            pl.BlockSpec((1, sq, h, d), lambda bi: (bi, 0, 0, 0)),
            pl.BlockSpec((1, skv, h, d), lambda bi: (bi, 0, 0, 0)),
            pl.BlockSpec((1, skv, h, d), lambda bi: (bi, 0, 0, 0)),
        ],
        out_specs=pl.BlockSpec((b, sq, h, d), lambda bi: (0, 0, 0, 0)),
        out_shape=jax.ShapeDtypeStruct((b, sq, h, d), jnp.float32),
        scratch_shapes=[
            pltpu.VMEM((b, h, sq, d), jnp.bfloat16),
            pltpu.VMEM((b, h, sq, d), jnp.bfloat16),
            pltpu.VMEM((b, 2, h, sq), jnp.float32),
            pltpu.VMEM((b, 2, h, sq), jnp.float32),
            pltpu.SemaphoreType.DMA((b,)),
            pltpu.SemaphoreType.DMA((b,)),
            pltpu.SemaphoreType.DMA((b,)),
            pltpu.SemaphoreType.DMA((b,)),
        ],
        compiler_params=pltpu.CompilerParams(
            collective_id=0,
            vmem_limit_bytes=100 * 1024 * 1024,
        ),
    )(Q, K, V)


# device time: 51642 ns/iter; 1.0067x vs baseline; 1.0067x over previous
import jax
import jax.numpy as jnp
from jax import lax
from jax.experimental import pallas as pl
from jax.experimental.pallas import tpu as pltpu


def _body(q_ref, k_ref, v_ref, out_ref,
          o_send, o_recv, st_send, st_recv,
          so_sems, ss_sems, ro_sems, rs_sems):
    nb = o_send.shape[0]
    d = q_ref.shape[-1]
    scale = d ** -0.5
    bi = pl.program_id(0)

    my_x = lax.axis_index("x")
    my_y = lax.axis_index("y")
    nbr = (my_x, 1 - my_y)

    @pl.when(bi == 0)
    def _barrier_signal():
        barrier = pltpu.get_barrier_semaphore()
        pl.semaphore_signal(
            barrier, inc=1, device_id=nbr,
            device_id_type=pl.DeviceIdType.MESH,
        )

    q = jnp.swapaxes(q_ref[0].astype(jnp.bfloat16), 0, 1)
    k = jnp.swapaxes(k_ref[0].astype(jnp.bfloat16), 0, 1)
    s = lax.dot_general(
        q, k, (((2,), (2,)), ((0,), (0,))),
        preferred_element_type=jnp.float32,
    ) * scale
    m = jnp.max(s, axis=-1, keepdims=True)
    p = jnp.exp(s - m)
    lsum = jnp.sum(p, axis=-1, keepdims=True)
    v = jnp.swapaxes(v_ref[0].astype(jnp.bfloat16), 0, 1)
    o = lax.dot_general(
        p.astype(jnp.bfloat16), v, (((2,), (1,)), ((0,), (0,))),
        preferred_element_type=jnp.float32,
    )
    o_send[bi] = o.astype(jnp.bfloat16)
    st_send[bi, 0] = m[..., 0]
    st_send[bi, 1] = lsum[..., 0]

    def _rdmas(idx):
        rdma_o = pltpu.make_async_remote_copy(
            src_ref=o_send.at[idx], dst_ref=o_recv.at[idx],
            send_sem=so_sems.at[idx], recv_sem=ro_sems.at[idx],
            device_id=nbr, device_id_type=pl.DeviceIdType.MESH,
        )
        rdma_st = pltpu.make_async_remote_copy(
            src_ref=st_send.at[idx], dst_ref=st_recv.at[idx],
            send_sem=ss_sems.at[idx], recv_sem=rs_sems.at[idx],
            device_id=nbr, device_id_type=pl.DeviceIdType.MESH,
        )
        return rdma_o, rdma_st

    @pl.when(bi == 0)
    def _barrier_wait():
        pl.semaphore_wait(pltpu.get_barrier_semaphore(), 1)

    rdma_o, rdma_st = _rdmas(bi)
    rdma_o.start()
    rdma_st.start()

    def _combine(j):
        ro, rs = _rdmas(j)
        ro.wait()
        rs.wait()
        m_loc = st_send[j, 0]
        l_loc = st_send[j, 1]
        m_nbr = st_recv[j, 0]
        l_nbr = st_recv[j, 1]
        m_new = jnp.maximum(m_loc, m_nbr)
        a_loc = jnp.exp(m_loc - m_new)
        a_nbr = jnp.exp(m_nbr - m_new)
        l_new = a_loc * l_loc + a_nbr * l_nbr
        o_loc = o_send[j].astype(jnp.float32)
        o_nbr = o_recv[j].astype(jnp.float32)
        num = a_loc[..., None] * o_loc + a_nbr[..., None] * o_nbr
        res = num / l_new[..., None]
        out_ref[j] = jnp.swapaxes(res, 0, 1)

    @pl.when(bi > 0)
    def _combine_prev():
        _combine(bi - 1)

    @pl.when(bi == nb - 1)
    def _combine_last():
        _combine(nb - 1)


def kernel(Q, K, V):
    b, sq, h, d = Q.shape
    skv = K.shape[1]

    return pl.pallas_call(
        _body,
        grid=(b,),
        in_specs=[
            pl.BlockSpec((1, sq, h, d), lambda bi: (bi, 0, 0, 0)),
            pl.BlockSpec((1, skv, h, d), lambda bi: (bi, 0, 0, 0)),
            pl.BlockSpec((1, skv, h, d), lambda bi: (bi, 0, 0, 0)),
        ],
        out_specs=pl.BlockSpec((b, sq, h, d), lambda bi: (0, 0, 0, 0)),
        out_shape=jax.ShapeDtypeStruct((b, sq, h, d), jnp.float32),
        scratch_shapes=[
            pltpu.VMEM((b, h, sq, d), jnp.bfloat16),
            pltpu.VMEM((b, h, sq, d), jnp.bfloat16),
            pltpu.VMEM((b, 2, h, sq), jnp.float32),
            pltpu.VMEM((b, 2, h, sq), jnp.float32),
            pltpu.SemaphoreType.DMA((b,)),
            pltpu.SemaphoreType.DMA((b,)),
            pltpu.SemaphoreType.DMA((b,)),
            pltpu.SemaphoreType.DMA((b,)),
        ],
        compiler_params=pltpu.CompilerParams(
            collective_id=0,
            vmem_limit_bytes=100 * 1024 * 1024,
        ),
    )(Q, K, V)


# device time: 51339 ns/iter; 1.0126x vs baseline; 1.0059x over previous
import jax
import jax.numpy as jnp
from jax import lax
from jax.experimental import pallas as pl
from jax.experimental.pallas import tpu as pltpu


def _body(q_ref, k_ref, v_ref, out_ref,
          o_send, o_recv, st_send, st_recv,
          so_sems, ss_sems, ro_sems, rs_sems):
    nb = o_send.shape[0]
    d = q_ref.shape[-1]
    scale = d ** -0.5
    bi = pl.program_id(0)

    my_x = lax.axis_index("x")
    my_y = lax.axis_index("y")
    nbr = (my_x, 1 - my_y)

    @pl.when(bi == 0)
    def _barrier_signal():
        barrier = pltpu.get_barrier_semaphore()
        pl.semaphore_signal(
            barrier, inc=1, device_id=nbr,
            device_id_type=pl.DeviceIdType.MESH,
        )

    q = jnp.swapaxes(q_ref[0].astype(jnp.bfloat16), 0, 1)
    k = k_ref[0].astype(jnp.bfloat16).reshape(16, -1, 128)
    s = lax.dot_general(
        q, k, (((2,), (2,)), ((0,), (0,))),
        preferred_element_type=jnp.float32,
    ) * scale
    m = jnp.max(s, axis=-1, keepdims=True)
    p = jnp.exp(s - m)
    lsum = jnp.sum(p, axis=-1, keepdims=True)
    v = v_ref[0].astype(jnp.bfloat16).reshape(16, -1, 128)
    o = lax.dot_general(
        p.astype(jnp.bfloat16), v, (((2,), (1,)), ((0,), (0,))),
        preferred_element_type=jnp.float32,
    )
    o_send[bi] = o.astype(jnp.bfloat16)
    st_send[bi, 0] = m[..., 0]
    st_send[bi, 1] = lsum[..., 0]

    def _rdmas(idx):
        rdma_o = pltpu.make_async_remote_copy(
            src_ref=o_send.at[idx], dst_ref=o_recv.at[idx],
            send_sem=so_sems.at[idx], recv_sem=ro_sems.at[idx],
            device_id=nbr, device_id_type=pl.DeviceIdType.MESH,
        )
        rdma_st = pltpu.make_async_remote_copy(
            src_ref=st_send.at[idx], dst_ref=st_recv.at[idx],
            send_sem=ss_sems.at[idx], recv_sem=rs_sems.at[idx],
            device_id=nbr, device_id_type=pl.DeviceIdType.MESH,
        )
        return rdma_o, rdma_st

    @pl.when(bi == 0)
    def _barrier_wait():
        pl.semaphore_wait(pltpu.get_barrier_semaphore(), 1)

    rdma_o, rdma_st = _rdmas(bi)
    rdma_o.start()
    rdma_st.start()

    def _combine(j):
        ro, rs = _rdmas(j)
        ro.wait()
        rs.wait()
        m_loc = st_send[j, 0]
        l_loc = st_send[j, 1]
        m_nbr = st_recv[j, 0]
        l_nbr = st_recv[j, 1]
        m_new = jnp.maximum(m_loc, m_nbr)
        a_loc = jnp.exp(m_loc - m_new)
        a_nbr = jnp.exp(m_nbr - m_new)
        l_new = a_loc * l_loc + a_nbr * l_nbr
        o_loc = o_send[j].astype(jnp.float32)
        o_nbr = o_recv[j].astype(jnp.float32)
        num = a_loc[..., None] * o_loc + a_nbr[..., None] * o_nbr
        res = num / l_new[..., None]
        out_ref[j] = jnp.swapaxes(res, 0, 1)

    @pl.when(bi > 0)
    def _combine_prev():
        _combine(bi - 1)

    @pl.when(bi == nb - 1)
    def _combine_last():
        _combine(nb - 1)


def kernel(Q, K, V):
    b, sq, h, d = Q.shape
    skv = K.shape[1]

    return pl.pallas_call(
        _body,
        grid=(b,),
        in_specs=[
            pl.BlockSpec((1, sq, h, d), lambda bi: (bi, 0, 0, 0)),
            pl.BlockSpec((1, skv, h, d), lambda bi: (bi, 0, 0, 0)),
            pl.BlockSpec((1, skv, h, d), lambda bi: (bi, 0, 0, 0)),
        ],
        out_specs=pl.BlockSpec((b, sq, h, d), lambda bi: (0, 0, 0, 0)),
        out_shape=jax.ShapeDtypeStruct((b, sq, h, d), jnp.float32),
        scratch_shapes=[
            pltpu.VMEM((b, h, sq, d), jnp.bfloat16),
            pltpu.VMEM((b, h, sq, d), jnp.bfloat16),
            pltpu.VMEM((b, 2, h, sq), jnp.float32),
            pltpu.VMEM((b, 2, h, sq), jnp.float32),
            pltpu.SemaphoreType.DMA((b,)),
            pltpu.SemaphoreType.DMA((b,)),
            pltpu.SemaphoreType.DMA((b,)),
            pltpu.SemaphoreType.DMA((b,)),
        ],
        compiler_params=pltpu.CompilerParams(
            collective_id=0,
            vmem_limit_bytes=100 * 1024 * 1024,
        ),
    )(Q, K, V)


# device time: 34245 ns/iter; 1.5181x vs baseline; 1.4992x over previous
import jax
import jax.numpy as jnp
from jax import lax
from jax.experimental import pallas as pl
from jax.experimental.pallas import tpu as pltpu


def _body(q_ref, k_ref, v_ref, out_ref,
          o_send, o_recv, st_send, st_recv, fin_send, fin_recv,
          so_sems, ss_sems, ro_sems, rs_sems, fs_sems, fr_sems):
    nb = o_send.shape[0]
    d = q_ref.shape[-1]
    scale = d ** -0.5
    bi = pl.program_id(0)

    my_x = lax.axis_index("x")
    my_y = lax.axis_index("y")
    ynbr = (my_x, 1 - my_y)
    xnbr = (1 - my_x, my_y)

    @pl.when(bi == 0)
    def _barrier_signal():
        barrier = pltpu.get_barrier_semaphore()
        for nbr in (ynbr, xnbr):
            pl.semaphore_signal(
                barrier, inc=1, device_id=nbr,
                device_id_type=pl.DeviceIdType.MESH,
            )

    q = jnp.swapaxes(q_ref[0].astype(jnp.bfloat16), 0, 1)
    k = jnp.swapaxes(k_ref[0].astype(jnp.bfloat16), 0, 1)
    s = lax.dot_general(
        q, k, (((2,), (2,)), ((0,), (0,))),
        preferred_element_type=jnp.float32,
    ) * scale
    m = jnp.max(s, axis=-1, keepdims=True)
    p = jnp.exp(s - m)
    lsum = jnp.sum(p, axis=-1, keepdims=True)
    v = jnp.swapaxes(v_ref[0].astype(jnp.bfloat16), 0, 1)
    o = lax.dot_general(
        p.astype(jnp.bfloat16), v, (((2,), (1,)), ((0,), (0,))),
        preferred_element_type=jnp.float32,
    )
    o_send[bi] = o.astype(jnp.bfloat16)
    st_send[bi, 0] = m[..., 0]
    st_send[bi, 1] = lsum[..., 0]

    def _part_rdmas(idx):
        rdma_o = pltpu.make_async_remote_copy(
            src_ref=o_send.at[idx], dst_ref=o_recv.at[idx],
            send_sem=so_sems.at[idx], recv_sem=ro_sems.at[idx],
            device_id=ynbr, device_id_type=pl.DeviceIdType.MESH,
        )
        rdma_st = pltpu.make_async_remote_copy(
            src_ref=st_send.at[idx], dst_ref=st_recv.at[idx],
            send_sem=ss_sems.at[idx], recv_sem=rs_sems.at[idx],
            device_id=ynbr, device_id_type=pl.DeviceIdType.MESH,
        )
        return rdma_o, rdma_st

    def _fin_rdma(idx):
        return pltpu.make_async_remote_copy(
            src_ref=fin_send.at[idx], dst_ref=fin_recv.at[idx],
            send_sem=fs_sems.at[idx], recv_sem=fr_sems.at[idx],
            device_id=xnbr, device_id_type=pl.DeviceIdType.MESH,
        )

    @pl.when(bi == 0)
    def _barrier_wait():
        pl.semaphore_wait(pltpu.get_barrier_semaphore(), 2)

    rdma_o, rdma_st = _part_rdmas(bi)
    rdma_o.start()
    rdma_st.start()

    def _combine(j):
        ro, rs = _part_rdmas(j)
        ro.wait()
        rs.wait()
        m_loc = st_send[j, 0]
        l_loc = st_send[j, 1]
        m_nbr = st_recv[j, 0]
        l_nbr = st_recv[j, 1]
        m_new = jnp.maximum(m_loc, m_nbr)
        a_loc = jnp.exp(m_loc - m_new)
        a_nbr = jnp.exp(m_nbr - m_new)
        l_new = a_loc * l_loc + a_nbr * l_nbr
        o_loc = o_send[j].astype(jnp.float32)
        o_nbr = o_recv[j].astype(jnp.float32)
        num = a_loc[..., None] * o_loc + a_nbr[..., None] * o_nbr
        res = jnp.swapaxes(num / l_new[..., None], 0, 1)
        out_ref[nb * my_x + j] = res
        fin_send[j] = res.astype(jnp.bfloat16)
        _fin_rdma(j).start()

    def _store_gathered(j):
        fr = _fin_rdma(j)
        fr.wait()
        out_ref[nb * (1 - my_x) + j] = fin_recv[j].astype(jnp.float32)

    @pl.when(bi > 0)
    def _combine_prev():
        _combine(bi - 1)

    @pl.when(bi > 1)
    def _store_prev():
        _store_gathered(bi - 2)

    @pl.when(bi == nb - 1)
    def _drain():
        _combine(nb - 1)
        _store_gathered(nb - 2)
        _store_gathered(nb - 1)


def kernel(Q, K, V):
    b, sq, h, d = Q.shape
    skv = K.shape[1]
    nb = b // 2

    def _in_idx(bi):
        return (nb * lax.axis_index("x") + bi, 0, 0, 0)

    return pl.pallas_call(
        _body,
        grid=(nb,),
        in_specs=[
            pl.BlockSpec((1, sq, h, d), _in_idx),
            pl.BlockSpec((1, skv, h, d), _in_idx),
            pl.BlockSpec((1, skv, h, d), _in_idx),
        ],
        out_specs=pl.BlockSpec((b, sq, h, d), lambda bi: (0, 0, 0, 0)),
        out_shape=jax.ShapeDtypeStruct((b, sq, h, d), jnp.float32),
        scratch_shapes=[
            pltpu.VMEM((nb, h, sq, d), jnp.bfloat16),
            pltpu.VMEM((nb, h, sq, d), jnp.bfloat16),
            pltpu.VMEM((nb, 2, h, sq), jnp.float32),
            pltpu.VMEM((nb, 2, h, sq), jnp.float32),
            pltpu.VMEM((nb, sq, h, d), jnp.bfloat16),
            pltpu.VMEM((nb, sq, h, d), jnp.bfloat16),
            pltpu.SemaphoreType.DMA((nb,)),
            pltpu.SemaphoreType.DMA((nb,)),
            pltpu.SemaphoreType.DMA((nb,)),
            pltpu.SemaphoreType.DMA((nb,)),
            pltpu.SemaphoreType.DMA((nb,)),
            pltpu.SemaphoreType.DMA((nb,)),
        ],
        compiler_params=pltpu.CompilerParams(
            collective_id=0,
            vmem_limit_bytes=100 * 1024 * 1024,
        ),
    )(Q, K, V)


# device time: 33385 ns/iter; 1.5572x vs baseline; 1.0258x over previous
import jax
import jax.numpy as jnp
from jax import lax
from jax.experimental import pallas as pl
from jax.experimental.pallas import tpu as pltpu


def _body(q_ref, k_ref, v_ref, out_ref,
          o_send, o_recv, st_send, st_recv, fin_send, fin_recv,
          so_sems, ss_sems, ro_sems, rs_sems, fs_sems, fr_sems):
    nt = o_send.shape[0]
    hh = o_send.shape[1]
    nb = nt // 2
    d = q_ref.shape[-1]
    scale = d ** -0.5
    t = pl.program_id(0)

    my_x = lax.axis_index("x")
    my_y = lax.axis_index("y")
    ynbr = (my_x, 1 - my_y)
    xnbr = (1 - my_x, my_y)

    @pl.when(t == 0)
    def _barrier_signal():
        barrier = pltpu.get_barrier_semaphore()
        for nbr in (ynbr, xnbr):
            pl.semaphore_signal(
                barrier, inc=1, device_id=nbr,
                device_id_type=pl.DeviceIdType.MESH,
            )

    q = jnp.swapaxes(q_ref[0].astype(jnp.bfloat16), 0, 1)
    k = jnp.swapaxes(k_ref[0].astype(jnp.bfloat16), 0, 1)
    s = lax.dot_general(
        q, k, (((2,), (2,)), ((0,), (0,))),
        preferred_element_type=jnp.float32,
    ) * scale
    m = jnp.max(s, axis=-1, keepdims=True)
    p = jnp.exp(s - m)
    lsum = jnp.sum(p, axis=-1, keepdims=True)
    v = jnp.swapaxes(v_ref[0].astype(jnp.bfloat16), 0, 1)
    o = lax.dot_general(
        p.astype(jnp.bfloat16), v, (((2,), (1,)), ((0,), (0,))),
        preferred_element_type=jnp.float32,
    )
    o_send[t] = o.astype(jnp.bfloat16)
    st_send[t, 0] = m[..., 0]
    st_send[t, 1] = lsum[..., 0]

    def _part_rdmas(idx):
        rdma_o = pltpu.make_async_remote_copy(
            src_ref=o_send.at[idx], dst_ref=o_recv.at[idx],
            send_sem=so_sems.at[idx], recv_sem=ro_sems.at[idx],
            device_id=ynbr, device_id_type=pl.DeviceIdType.MESH,
        )
        rdma_st = pltpu.make_async_remote_copy(
            src_ref=st_send.at[idx], dst_ref=st_recv.at[idx],
            send_sem=ss_sems.at[idx], recv_sem=rs_sems.at[idx],
            device_id=ynbr, device_id_type=pl.DeviceIdType.MESH,
        )
        return rdma_o, rdma_st

    def _fin_rdma(idx):
        return pltpu.make_async_remote_copy(
            src_ref=fin_send.at[idx], dst_ref=fin_recv.at[idx],
            send_sem=fs_sems.at[idx], recv_sem=fr_sems.at[idx],
            device_id=xnbr, device_id_type=pl.DeviceIdType.MESH,
        )

    @pl.when(t == 0)
    def _barrier_wait():
        pl.semaphore_wait(pltpu.get_barrier_semaphore(), 2)

    rdma_o, rdma_st = _part_rdmas(t)
    rdma_o.start()
    rdma_st.start()

    def _combine(j):
        ro, rs = _part_rdmas(j)
        ro.wait()
        rs.wait()
        m_loc = st_send[j, 0]
        l_loc = st_send[j, 1]
        m_nbr = st_recv[j, 0]
        l_nbr = st_recv[j, 1]
        m_new = jnp.maximum(m_loc, m_nbr)
        a_loc = jnp.exp(m_loc - m_new)
        a_nbr = jnp.exp(m_nbr - m_new)
        l_new = a_loc * l_loc + a_nbr * l_nbr
        o_loc = o_send[j].astype(jnp.float32)
        o_nbr = o_recv[j].astype(jnp.float32)
        num = a_loc[..., None] * o_loc + a_nbr[..., None] * o_nbr
        res = jnp.swapaxes(num / l_new[..., None], 0, 1)
        g = nb * my_x + j // 2
        out_ref[g, :, pl.ds(hh * (j % 2), hh), :] = res
        fin_send[j] = res.astype(jnp.bfloat16)
        _fin_rdma(j).start()

    def _store_gathered(j):
        fr = _fin_rdma(j)
        fr.wait()
        g = nb * (1 - my_x) + j // 2
        out_ref[g, :, pl.ds(hh * (j % 2), hh), :] = (
            fin_recv[j].astype(jnp.float32))

    @pl.when(t > 1)
    def _combine_prev():
        _combine(t - 2)

    @pl.when(t > 3)
    def _store_prev():
        _store_gathered(t - 4)

    @pl.when(t == nt - 1)
    def _drain():
        _combine(nt - 2)
        _combine(nt - 1)
        for j in range(nt - 4, nt):
            _store_gathered(j)


def kernel(Q, K, V):
    b, sq, h, d = Q.shape
    skv = K.shape[1]
    nb = b // 2
    hh = h // 2
    nt = 2 * nb

    def _in_idx(t):
        return (nb * lax.axis_index("x") + t // 2, 0, t % 2, 0)

    return pl.pallas_call(
        _body,
        grid=(nt,),
        in_specs=[
            pl.BlockSpec((1, sq, hh, d), _in_idx),
            pl.BlockSpec((1, skv, hh, d), _in_idx),
            pl.BlockSpec((1, skv, hh, d), _in_idx),
        ],
        out_specs=pl.BlockSpec((b, sq, h, d), lambda t: (0, 0, 0, 0)),
        out_shape=jax.ShapeDtypeStruct((b, sq, h, d), jnp.float32),
        scratch_shapes=[
            pltpu.VMEM((nt, hh, sq, d), jnp.bfloat16),
            pltpu.VMEM((nt, hh, sq, d), jnp.bfloat16),
            pltpu.VMEM((nt, 2, hh, sq), jnp.float32),
            pltpu.VMEM((nt, 2, hh, sq), jnp.float32),
            pltpu.VMEM((nt, sq, hh, d), jnp.bfloat16),
            pltpu.VMEM((nt, sq, hh, d), jnp.bfloat16),
            pltpu.SemaphoreType.DMA((nt,)),
            pltpu.SemaphoreType.DMA((nt,)),
            pltpu.SemaphoreType.DMA((nt,)),
            pltpu.SemaphoreType.DMA((nt,)),
            pltpu.SemaphoreType.DMA((nt,)),
            pltpu.SemaphoreType.DMA((nt,)),
        ],
        compiler_params=pltpu.CompilerParams(
            collective_id=0,
            vmem_limit_bytes=100 * 1024 * 1024,
        ),
    )(Q, K, V)


# device time: 32777 ns/iter; 1.5861x vs baseline; 1.0185x over previous
import jax
import jax.numpy as jnp
from jax import lax
from jax.experimental import pallas as pl
from jax.experimental.pallas import tpu as pltpu


def _body(q_ref, k_ref, v_ref, out_ref,
          o_send, st_send, o_rcv_y, st_rcv_y, o_rcv_x, st_rcv_x,
          o_rcv_d, st_rcv_d,
          os_sems, ss_sems, or_sems, sr_sems):
    nt = o_send.shape[0]
    hh = o_send.shape[1]
    nb = nt // 2
    d = q_ref.shape[-1]
    scale = d ** -0.5
    t = pl.program_id(0)

    my_x = lax.axis_index("x")
    my_y = lax.axis_index("y")
    ynbr = (my_x, 1 - my_y)
    xnbr = (1 - my_x, my_y)
    dnbr = (1 - my_x, 1 - my_y)
    peers = (ynbr, xnbr, dnbr)
    o_rcvs = (o_rcv_y, o_rcv_x, o_rcv_d)
    st_rcvs = (st_rcv_y, st_rcv_x, st_rcv_d)

    @pl.when(t == 0)
    def _barrier_signal():
        barrier = pltpu.get_barrier_semaphore()
        for nbr in peers:
            pl.semaphore_signal(
                barrier, inc=1, device_id=nbr,
                device_id_type=pl.DeviceIdType.MESH,
            )

    q = jnp.swapaxes(q_ref[0].astype(jnp.bfloat16), 0, 1)
    k = jnp.swapaxes(k_ref[0].astype(jnp.bfloat16), 0, 1)
    s = lax.dot_general(
        q, k, (((2,), (2,)), ((0,), (0,))),
        preferred_element_type=jnp.float32,
    ) * scale
    m = jnp.max(s, axis=-1, keepdims=True)
    p = jnp.exp(s - m)
    lsum = jnp.sum(p, axis=-1, keepdims=True)
    v = jnp.swapaxes(v_ref[0].astype(jnp.bfloat16), 0, 1)
    o = lax.dot_general(
        p.astype(jnp.bfloat16), v, (((2,), (1,)), ((0,), (0,))),
        preferred_element_type=jnp.float32,
    )
    o_send[t] = o.astype(jnp.bfloat16)
    st_send[t, 0] = m[..., 0]
    st_send[t, 1] = lsum[..., 0]

    def _rdmas(idx, pi):
        rdma_o = pltpu.make_async_remote_copy(
            src_ref=o_send.at[idx], dst_ref=o_rcvs[pi].at[idx],
            send_sem=os_sems.at[pi, idx], recv_sem=or_sems.at[pi, idx],
            device_id=peers[pi], device_id_type=pl.DeviceIdType.MESH,
        )
        rdma_st = pltpu.make_async_remote_copy(
            src_ref=st_send.at[idx], dst_ref=st_rcvs[pi].at[idx],
            send_sem=ss_sems.at[pi, idx], recv_sem=sr_sems.at[pi, idx],
            device_id=peers[pi], device_id_type=pl.DeviceIdType.MESH,
        )
        return rdma_o, rdma_st

    @pl.when(t == 0)
    def _barrier_wait():
        pl.semaphore_wait(pltpu.get_barrier_semaphore(), 3)

    for pi in range(3):
        rdma_o, rdma_st = _rdmas(t, pi)
        rdma_o.start()
        rdma_st.start()

    def _merge(j, o_a, m_a, l_a, o_b, m_b, l_b, g):
        m_new = jnp.maximum(m_a, m_b)
        a_a = jnp.exp(m_a - m_new)
        a_b = jnp.exp(m_b - m_new)
        l_new = a_a * l_a + a_b * l_b
        num = (a_a[..., None] * o_a.astype(jnp.float32)
               + a_b[..., None] * o_b.astype(jnp.float32))
        res = jnp.swapaxes(num / l_new[..., None], 0, 1)
        out_ref[g, :, pl.ds(hh * (j % 2), hh), :] = res

    def _combine(j):
        for pi in range(3):
            ro, rs = _rdmas(j, pi)
            ro.wait()
            rs.wait()
        _merge(j, o_send[j], st_send[j, 0], st_send[j, 1],
               o_rcv_y[j], st_rcv_y[j, 0], st_rcv_y[j, 1],
               nb * my_x + j // 2)
        _merge(j, o_rcv_x[j], st_rcv_x[j, 0], st_rcv_x[j, 1],
               o_rcv_d[j], st_rcv_d[j, 0], st_rcv_d[j, 1],
               nb * (1 - my_x) + j // 2)

    @pl.when(t > 1)
    def _combine_prev():
        _combine(t - 2)

    @pl.when(t == nt - 1)
    def _drain():
        _combine(nt - 2)
        _combine(nt - 1)


def kernel(Q, K, V):
    b, sq, h, d = Q.shape
    skv = K.shape[1]
    nb = b // 2
    hh = h // 2
    nt = 2 * nb

    def _in_idx(t):
        return (nb * lax.axis_index("x") + t // 2, 0, t % 2, 0)

    recv_o = pltpu.VMEM((nt, hh, sq, d), jnp.bfloat16)
    recv_st = pltpu.VMEM((nt, 2, hh, sq), jnp.float32)

    return pl.pallas_call(
        _body,
        grid=(nt,),
        in_specs=[
            pl.BlockSpec((1, sq, hh, d), _in_idx),
            pl.BlockSpec((1, skv, hh, d), _in_idx),
            pl.BlockSpec((1, skv, hh, d), _in_idx),
        ],
        out_specs=pl.BlockSpec((b, sq, h, d), lambda t: (0, 0, 0, 0)),
        out_shape=jax.ShapeDtypeStruct((b, sq, h, d), jnp.float32),
        scratch_shapes=[
            pltpu.VMEM((nt, hh, sq, d), jnp.bfloat16),
            pltpu.VMEM((nt, 2, hh, sq), jnp.float32),
            recv_o, recv_st,
            recv_o, recv_st,
            recv_o, recv_st,
            pltpu.SemaphoreType.DMA((3, nt)),
            pltpu.SemaphoreType.DMA((3, nt)),
            pltpu.SemaphoreType.DMA((3, nt)),
            pltpu.SemaphoreType.DMA((3, nt)),
        ],
        compiler_params=pltpu.CompilerParams(
            collective_id=0,
            vmem_limit_bytes=100 * 1024 * 1024,
        ),
    )(Q, K, V)
